# baseline (speedup 1.0000x reference)
import functools

import jax
import jax.numpy as jnp
from jax import lax
from jax.experimental import pallas as pl
from jax.experimental.pallas import tpu as pltpu
from jax.experimental.pallas import tpu_sc as plsc

_DIM = 128
_CHUNK = 128


@functools.partial(jax.jit, static_argnames=())
def _gather(timestep, pos_embd_matrix):
    info = plsc.get_sparse_core_info()
    nw = info.num_cores * info.num_subcores
    batch = timestep.shape[0]
    dim = pos_embd_matrix.shape[1]
    b_per_w = batch // nw
    n_chunks = b_per_w // _CHUNK

    idx3 = timestep.reshape(nw, n_chunks, _CHUNK)
    mesh = plsc.VectorSubcoreMesh(core_axis_name="c", subcore_axis_name="s")

    @functools.partial(
        pl.kernel,
        mesh=mesh,
        out_type=jax.ShapeDtypeStruct((batch, dim), jnp.float32),
        scratch_types=[
            pltpu.VMEM((n_chunks, _CHUNK), jnp.int32),
            pltpu.VMEM((b_per_w, dim), jnp.float32),
            pltpu.SemaphoreType.DMA((n_chunks,)),
            pltpu.SemaphoreType.DMA,
        ],
    )
    def k(idx_hbm, table_hbm, out_hbm, idx_v, rows_v, gsems, wsem):
        wid = lax.axis_index("s") * info.num_cores + lax.axis_index("c")
        base = wid * b_per_w
        pltpu.sync_copy(idx_hbm.at[wid], idx_v)
        gathers = []
        for j in range(n_chunks):
            c = pltpu.make_async_copy(
                table_hbm.at[idx_v.at[j]],
                rows_v.at[pl.ds(j * _CHUNK, _CHUNK)],
                gsems.at[j],
            )
            c.start()
            gathers.append(c)
        writes = []
        for j in range(n_chunks):
            gathers[j].wait()
            w = pltpu.make_async_copy(
                rows_v.at[pl.ds(j * _CHUNK, _CHUNK)],
                out_hbm.at[pl.ds(base + j * _CHUNK, _CHUNK)],
                wsem,
            )
            w.start()
            writes.append(w)
        for w in writes:
            w.wait()

    return k(idx3, pos_embd_matrix)


def kernel(timestep, pos_embd_matrix):
    return _gather(timestep, pos_embd_matrix)


# speedup vs baseline: 1.1735x; 1.1735x over previous
import functools

import jax
import jax.numpy as jnp
from jax import lax
from jax.experimental import pallas as pl
from jax.experimental.pallas import tpu as pltpu
from jax.experimental.pallas import tpu_sc as plsc

_DIM = 128
_CHUNK = 128


@functools.partial(jax.jit, static_argnames=())
def _gather(timestep, pos_embd_matrix):
    info = plsc.get_sparse_core_info()
    nw = info.num_cores * info.num_subcores
    batch = timestep.shape[0]
    dim = pos_embd_matrix.shape[1]
    b_per_w = batch // nw
    n_chunks = b_per_w // _CHUNK

    vocab = pos_embd_matrix.shape[0]
    n_stagers = 5
    rows_per_stager = vocab // n_stagers
    idx3 = timestep.reshape(nw, n_chunks, _CHUNK)
    mesh = plsc.VectorSubcoreMesh(core_axis_name="c", subcore_axis_name="s")

    @functools.partial(
        pl.kernel,
        mesh=mesh,
        out_type=jax.ShapeDtypeStruct((batch, dim), jnp.float32),
        scratch_types=[
            pltpu.VMEM((n_chunks, _CHUNK), jnp.int32),
            pltpu.VMEM((b_per_w, dim), jnp.float32),
            pltpu.VMEM_SHARED((vocab, dim), jnp.float32),
            pltpu.SemaphoreType.DMA((n_chunks,)),
            pltpu.SemaphoreType.DMA,
        ],
    )
    def k(idx_hbm, table_hbm, out_hbm, idx_v, rows_v, table_sp, gsems, wsem):
        sid = lax.axis_index("s")
        wid = sid * info.num_cores + lax.axis_index("c")
        base = wid * b_per_w
        pltpu.sync_copy(idx_hbm.at[wid], idx_v)
        @pl.when(sid < n_stagers)
        def _stage():
            pltpu.sync_copy(
                table_hbm.at[pl.ds(sid * rows_per_stager, rows_per_stager)],
                table_sp.at[pl.ds(sid * rows_per_stager, rows_per_stager)],
            )
        plsc.subcore_barrier()
        gathers = []
        for j in range(n_chunks):
            c = pltpu.make_async_copy(
                table_sp.at[idx_v.at[j]],
                rows_v.at[pl.ds(j * _CHUNK, _CHUNK)],
                gsems.at[j],
            )
            c.start()
            gathers.append(c)
        writes = []
        for j in range(n_chunks):
            gathers[j].wait()
            w = pltpu.make_async_copy(
                rows_v.at[pl.ds(j * _CHUNK, _CHUNK)],
                out_hbm.at[pl.ds(base + j * _CHUNK, _CHUNK)],
                wsem,
            )
            w.start()
            writes.append(w)
        for w in writes:
            w.wait()

    return k(idx3, pos_embd_matrix)


def kernel(timestep, pos_embd_matrix):
    return _gather(timestep, pos_embd_matrix)


# speedup vs baseline: 1.2001x; 1.0227x over previous
import functools

import jax
import jax.numpy as jnp
from jax import lax
from jax.experimental import pallas as pl
from jax.experimental.pallas import tpu as pltpu
from jax.experimental.pallas import tpu_sc as plsc

_DIM = 128
_CHUNK = 64


@functools.partial(jax.jit, static_argnames=())
def _gather(timestep, pos_embd_matrix):
    info = plsc.get_sparse_core_info()
    nw = info.num_cores * info.num_subcores
    batch = timestep.shape[0]
    dim = pos_embd_matrix.shape[1]
    b_per_w = batch // nw
    n_chunks = b_per_w // _CHUNK

    vocab = pos_embd_matrix.shape[0]
    n_stagers = 5
    rows_per_stager = vocab // n_stagers
    idx3 = timestep.reshape(nw, n_chunks, _CHUNK)
    mesh = plsc.VectorSubcoreMesh(core_axis_name="c", subcore_axis_name="s")

    @functools.partial(
        pl.kernel,
        mesh=mesh,
        out_type=jax.ShapeDtypeStruct((batch, dim), jnp.float32),
        scratch_types=[
            pltpu.VMEM((n_chunks, _CHUNK), jnp.int32),
            pltpu.VMEM((b_per_w, dim), jnp.float32),
            pltpu.VMEM_SHARED((vocab, dim), jnp.float32),
            pltpu.SemaphoreType.DMA((n_chunks,)),
            pltpu.SemaphoreType.DMA,
            pltpu.SemaphoreType.DMA,
        ],
    )
    def k(idx_hbm, table_hbm, out_hbm, idx_v, rows_v, table_sp, gsems, wsem, ssem):
        sid = lax.axis_index("s")
        wid = sid * info.num_cores + lax.axis_index("c")
        base = wid * b_per_w
        @pl.when(sid < n_stagers)
        def _stage():
            pltpu.make_async_copy(
                table_hbm.at[pl.ds(sid * rows_per_stager, rows_per_stager)],
                table_sp.at[pl.ds(sid * rows_per_stager, rows_per_stager)],
                ssem,
            ).start()
        pltpu.sync_copy(idx_hbm.at[wid], idx_v)
        @pl.when(sid < n_stagers)
        def _stage_wait():
            pltpu.make_async_copy(
                table_hbm.at[pl.ds(sid * rows_per_stager, rows_per_stager)],
                table_sp.at[pl.ds(sid * rows_per_stager, rows_per_stager)],
                ssem,
            ).wait()
        plsc.subcore_barrier()
        gathers = []
        for j in range(n_chunks):
            c = pltpu.make_async_copy(
                table_sp.at[idx_v.at[j]],
                rows_v.at[pl.ds(j * _CHUNK, _CHUNK)],
                gsems.at[j],
            )
            c.start()
            gathers.append(c)
        writes = []
        for j in range(n_chunks):
            gathers[j].wait()
            w = pltpu.make_async_copy(
                rows_v.at[pl.ds(j * _CHUNK, _CHUNK)],
                out_hbm.at[pl.ds(base + j * _CHUNK, _CHUNK)],
                wsem,
            )
            w.start()
            writes.append(w)
        for w in writes:
            w.wait()

    return k(idx3, pos_embd_matrix)


def kernel(timestep, pos_embd_matrix):
    return _gather(timestep, pos_embd_matrix)


# speedup vs baseline: 1.2034x; 1.0028x over previous
import functools

import jax
import jax.numpy as jnp
from jax import lax
from jax.experimental import pallas as pl
from jax.experimental.pallas import tpu as pltpu
from jax.experimental.pallas import tpu_sc as plsc

_DIM = 128
_CHUNK = 64


@functools.partial(jax.jit, static_argnames=())
def _gather(timestep, pos_embd_matrix):
    info = plsc.get_sparse_core_info()
    nw = info.num_cores * info.num_subcores
    batch = timestep.shape[0]
    dim = pos_embd_matrix.shape[1]
    b_per_w = batch // nw
    n_chunks = b_per_w // _CHUNK

    vocab = pos_embd_matrix.shape[0]
    stage_step = 64
    stage_tail = vocab - 15 * stage_step
    idx3 = timestep.reshape(nw, n_chunks, _CHUNK)
    mesh = plsc.VectorSubcoreMesh(core_axis_name="c", subcore_axis_name="s")

    @functools.partial(
        pl.kernel,
        mesh=mesh,
        out_type=jax.ShapeDtypeStruct((batch, dim), jnp.float32),
        scratch_types=[
            pltpu.VMEM((n_chunks, _CHUNK), jnp.int32),
            pltpu.VMEM((b_per_w, dim), jnp.float32),
            pltpu.VMEM_SHARED((vocab, dim), jnp.float32),
            pltpu.SemaphoreType.DMA((n_chunks,)),
            pltpu.SemaphoreType.DMA,
            pltpu.SemaphoreType.DMA,
        ],
    )
    def k(idx_hbm, table_hbm, out_hbm, idx_v, rows_v, table_sp, gsems, wsem, ssem):
        sid = lax.axis_index("s")
        wid = sid * info.num_cores + lax.axis_index("c")
        base = wid * b_per_w
        @pl.when(sid < 15)
        def _stage_body():
            pltpu.make_async_copy(
                table_hbm.at[pl.ds(sid * stage_step, stage_step)],
                table_sp.at[pl.ds(sid * stage_step, stage_step)],
                ssem,
            ).start()
        @pl.when(sid == 15)
        def _stage_tail():
            pltpu.make_async_copy(
                table_hbm.at[pl.ds(15 * stage_step, stage_tail)],
                table_sp.at[pl.ds(15 * stage_step, stage_tail)],
                ssem,
            ).start()
        pltpu.sync_copy(idx_hbm.at[wid], idx_v)
        @pl.when(sid < 15)
        def _stage_body_wait():
            pltpu.make_async_copy(
                table_hbm.at[pl.ds(sid * stage_step, stage_step)],
                table_sp.at[pl.ds(sid * stage_step, stage_step)],
                ssem,
            ).wait()
        @pl.when(sid == 15)
        def _stage_tail_wait():
            pltpu.make_async_copy(
                table_hbm.at[pl.ds(15 * stage_step, stage_tail)],
                table_sp.at[pl.ds(15 * stage_step, stage_tail)],
                ssem,
            ).wait()
        plsc.subcore_barrier()
        gathers = []
        for j in range(n_chunks):
            c = pltpu.make_async_copy(
                table_sp.at[idx_v.at[j]],
                rows_v.at[pl.ds(j * _CHUNK, _CHUNK)],
                gsems.at[j],
            )
            c.start()
            gathers.append(c)
        writes = []
        for j in range(n_chunks):
            gathers[j].wait()
            w = pltpu.make_async_copy(
                rows_v.at[pl.ds(j * _CHUNK, _CHUNK)],
                out_hbm.at[pl.ds(base + j * _CHUNK, _CHUNK)],
                wsem,
            )
            w.start()
            writes.append(w)
        for w in writes:
            w.wait()

    return k(idx3, pos_embd_matrix)


def kernel(timestep, pos_embd_matrix):
    return _gather(timestep, pos_embd_matrix)
